# baseline (speedup 1.0000x reference)
import functools

import jax
import jax.numpy as jnp
from jax import lax
from jax.experimental import pallas as pl
from jax.experimental.pallas import tpu as pltpu
from jax.experimental.pallas import tpu_sc as plsc

B, S, D = 4, 8192, 1024
NC, NS = 2, 16
NW = NC * NS
S_PER_W = S // NW
CHUNK = 8
N_CHUNKS = S_PER_W // CHUNK
LANES = 16


def _sc_body(x_hbm, pe_hbm, out_hbm,
             peb0, peb1,
             xb00, xb01, xb02, xb03,
             xb10, xb11, xb12, xb13,
             pe_sem0, pe_sem1, in_sem0, in_sem1, out_sem):
    peb = (peb0, peb1)
    xb = ((xb00, xb01, xb02, xb03), (xb10, xb11, xb12, xb13))
    pe_sem = (pe_sem0, pe_sem1)
    in_sem = (in_sem0, in_sem1)

    wid = lax.axis_index("s") * NC + lax.axis_index("c")
    base = wid * S_PER_W

    def pe_row(c):
        return pl.multiple_of(base + c * CHUNK, 8)

    def x_row(c, b):
        return pl.multiple_of(b * S + base + c * CHUNK, 8)

    def issue_pe(c, p):
        pltpu.async_copy(pe_hbm.at[pl.ds(pe_row(c), CHUNK)], peb[p], pe_sem[p])

    def issue_in(c, p):
        for b in range(B):
            pltpu.async_copy(x_hbm.at[pl.ds(x_row(c, b), CHUNK)], xb[p][b],
                             in_sem[p])

    def wait_pe(p):
        pltpu.make_async_copy(pe_hbm.at[pl.ds(0, CHUNK)], peb[p],
                              pe_sem[p]).wait()

    def wait_in(p):
        for b in range(B):
            pltpu.make_async_copy(x_hbm.at[pl.ds(0, CHUNK)], xb[p][b],
                                  in_sem[p]).wait()

    def drain_outs():
        for b in range(B):
            pltpu.make_async_copy(x_hbm.at[pl.ds(0, CHUNK)], xb[0][b],
                                  out_sem).wait()

    def chunk_step(c, p):
        wait_pe(p)

        @pl.when(c + 1 < N_CHUNKS)
        def _():
            issue_pe(c + 1, 1 - p)

        wait_in(p)

        @pl.when(c > 0)
        def _():
            drain_outs()

        @pl.when(c + 1 < N_CHUNKS)
        def _():
            issue_in(c + 1, 1 - p)

        bufs = xb[p]
        pbuf = peb[p]

        for r in range(CHUNK):
            @plsc.parallel_loop(0, D, LANES, unroll=8)
            def _(i, r=r):
                j = pl.multiple_of(i, 8)
                pe_slice = pbuf[r, pl.ds(j, LANES)]
                for b in range(B):
                    plsc.addupdate(bufs[b].at[r, pl.ds(j, LANES)], pe_slice)

        for b in range(B):
            pltpu.async_copy(bufs[b], out_hbm.at[pl.ds(x_row(c, b), CHUNK)],
                             out_sem)

    issue_pe(0, 0)
    issue_in(0, 0)

    def loop_body(t, carry):
        chunk_step(2 * t, 0)
        chunk_step(2 * t + 1, 1)
        return carry

    lax.fori_loop(0, N_CHUNKS // 2, loop_body, 0)
    drain_outs()


@jax.jit
def kernel(x, pe_table):
    mesh = plsc.VectorSubcoreMesh(core_axis_name="c", subcore_axis_name="s")
    k = functools.partial(
        pl.kernel,
        mesh=mesh,
        out_type=jax.ShapeDtypeStruct((B * S, D), jnp.float32),
        scratch_types=(
            [pltpu.VMEM((CHUNK, D), jnp.float32)] * 2
            + [pltpu.VMEM((CHUNK, D), jnp.float32)] * 8
            + [pltpu.SemaphoreType.DMA] * 5
        ),
    )(_sc_body)
    out2d = k(x.reshape(B * S, D), pe_table)
    return out2d.reshape(B, S, D)
